# baseline (speedup 1.0000x reference)
import functools
import jax
import jax.numpy as jnp
from jax import lax
from jax.experimental import pallas as pl
from jax.experimental.pallas import tpu as pltpu, tpu_sc as plsc

_BASE_VOCAB_SHIFT = 100
_COORD_VOCAB_SHIFT = 1000
_QUANT_BINS = 1000
_MAX_INPUT_SIZE = 1024.0

_B, _S, _V = 32, 500, 3000
_N = _S // 5
_NP = 104

_W = 8

_NC = 2


@functools.cache
def _sc_gather_rows_fn():
    mesh = plsc.VectorSubcoreMesh(core_axis_name="c", subcore_axis_name="s")

    @functools.partial(
        pl.kernel, mesh=mesh,
        out_type=jax.ShapeDtypeStruct((_B * _NP, _V), jnp.float32),
        scratch_types=[
            pltpu.VMEM((_W, _V), jnp.float32),
            pltpu.VMEM((_W, _V), jnp.float32),
            pltpu.SemaphoreType.DMA,
            pltpu.SemaphoreType.DMA,
            pltpu.SemaphoreType.DMA,
            pltpu.SemaphoreType.DMA,
        ],
    )
    def _sc_gather_rows(x_hbm, out_hbm, rows_a, rows_b,
                        gsem_a, gsem_b, wsem_a, wsem_b):
        wid = lax.axis_index("s") * _NC + lax.axis_index("c")
        bufs = (rows_a, rows_b)
        gsems = (gsem_a, gsem_b)
        wsems = (wsem_a, wsem_b)
        nwave = -(-_N // _W)

        def start_gathers(g, buf, sem):
            base = g * _W
            return [
                pltpu.async_copy(x_hbm.at[wid, 5 * (base + k) + 4],
                                 buf.at[k], sem)
                for k in range(min(_W, _N - base))
            ]

        def start_write(g, buf, sem):
            dst = out_hbm.at[pl.ds(_NP * wid + _W * g, _W)]
            return [pltpu.async_copy(buf, dst, sem)]

        pend_g = start_gathers(0, bufs[0], gsems[0])
        pend_w = []
        for g in range(nwave):
            cur = g % 2
            for h in pend_g:
                h.wait()
            for h in pend_w:
                h.wait()
            if g + 1 < nwave:
                pend_g = start_gathers(g + 1, bufs[1 - cur], gsems[1 - cur])
            pend_w = start_write(g, bufs[cur], wsems[cur])
        for h in pend_w:
            h.wait()

    return _sc_gather_rows


def _decode_body(x_ref, seq_ref, orig_ref, size_ref,
                 cls_ref, bbox_ref, score_ref):
    x = x_ref[...]
    rows, vp = x.shape
    col = lax.broadcasted_iota(jnp.int32, (1, vp), 1)
    m = jnp.max(x, axis=-1, keepdims=True)
    denom = jnp.sum(jnp.exp(x - m), axis=-1, keepdims=True)
    inwin = (col >= _BASE_VOCAB_SHIFT) & (col < _COORD_VOCAB_SHIFT)
    xm = jnp.where(inwin, x, -jnp.inf)
    mw = jnp.max(xm, axis=-1, keepdims=True)
    idx = jnp.argmax(xm, axis=-1, keepdims=True)
    cls = jnp.maximum(idx - _BASE_VOCAB_SHIFT, 0)
    score = jnp.exp(mw - m) / denom
    cls_ref[0] = cls[:_N]
    score_ref[0] = score[:_N]

    sq = seq_ref[0]
    q = (sq - _COORD_VOCAB_SHIFT).astype(jnp.float32) / (_QUANT_BINS - 1)
    q = jnp.clip(q, 0.0, 1.0)
    sc = (_MAX_INPUT_SIZE / size_ref[0]) * orig_ref[0]
    bbox = jnp.concatenate(
        [q[:, 1:2], q[:, 0:1], q[:, 3:4], q[:, 2:3]], axis=1)
    scl4 = jnp.concatenate(
        [sc[:, 0:1], sc[:, 1:2], sc[:, 0:1], sc[:, 1:2]], axis=1)
    bbox_ref[0] = bbox * scl4


def kernel(pred_seq_logits, pred_seq, orig_size, size, image_id):
    b, s, v = pred_seq_logits.shape
    n = s // 5
    compact = _sc_gather_rows_fn()(pred_seq_logits)

    seq3 = pred_seq.reshape(b, n, 5)
    orig_f = orig_size.astype(jnp.float32).reshape(b, 1, 2)
    size_f = size.astype(jnp.float32).reshape(b, 1, 2)

    cls, bbox, score = pl.pallas_call(
        _decode_body,
        grid=(b,),
        in_specs=[
            pl.BlockSpec((_NP, _V), lambda i: (i, 0)),
            pl.BlockSpec((1, n, 5), lambda i: (i, 0, 0)),
            pl.BlockSpec((1, 1, 2), lambda i: (i, 0, 0)),
            pl.BlockSpec((1, 1, 2), lambda i: (i, 0, 0)),
        ],
        out_specs=[
            pl.BlockSpec((1, n, 1), lambda i: (i, 0, 0)),
            pl.BlockSpec((1, n, 4), lambda i: (i, 0, 0)),
            pl.BlockSpec((1, n, 1), lambda i: (i, 0, 0)),
        ],
        out_shape=[
            jax.ShapeDtypeStruct((b, n, 1), jnp.int32),
            jax.ShapeDtypeStruct((b, n, 4), jnp.float32),
            jax.ShapeDtypeStruct((b, n, 1), jnp.float32),
        ],
        compiler_params=pltpu.CompilerParams(
            dimension_semantics=("arbitrary",)),
    )(compact, seq3, orig_f, size_f)
    return cls[..., 0], bbox, score[..., 0]
